# baseline (speedup 1.0000x reference)
import functools

import jax
import jax.numpy as jnp
from jax import lax
from jax.experimental import pallas as pl
from jax.experimental.pallas import tpu as pltpu
from jax.experimental.pallas import tpu_sc as plsc

N = 10000
E = 320000
D = 128
NC = 2
NS = 16
NP = 10240
EP = 327680
CH = EP // (NC * NS * 128)
RT = NP // NS

_mesh = plsc.VectorSubcoreMesh(
    core_axis_name="c", subcore_axis_name="s", num_cores=NC, num_subcores=NS
)


@functools.partial(
    pl.kernel,
    out_type=jax.ShapeDtypeStruct((NC, NP), jnp.float32),
    mesh=_mesh,
    scratch_types=[
        pltpu.VMEM((CH, 128), jnp.int32),
        pltpu.VMEM((128,), jnp.float32),
        pltpu.VMEM_SHARED((NP,), jnp.float32),
        pltpu.SemaphoreType.DMA,
    ],
)
def _sc_degree(dst_hbm, ones_hbm, zeros_hbm, deg_hbm, dstv, ones_v, deg_sh, dsem):
    c = lax.axis_index("c")
    s = lax.axis_index("s")
    pltpu.sync_copy(zeros_hbm.at[pl.ds(s * RT, RT)], deg_sh.at[pl.ds(s * RT, RT)])
    pltpu.sync_copy(dst_hbm.at[c, s], dstv)
    pltpu.sync_copy(ones_hbm, ones_v)
    plsc.subcore_barrier()

    def fire(j, carry):
        pltpu.async_copy(ones_v, deg_sh.at[dstv.at[j]], dsem, add=True)
        return carry

    lax.fori_loop(0, CH, fire, 0)

    def drain(j, carry):
        pltpu.make_async_copy(ones_v, deg_sh.at[dstv.at[j]], dsem).wait()
        return carry

    lax.fori_loop(0, CH, drain, 0)
    plsc.subcore_barrier()
    pltpu.sync_copy(deg_sh.at[pl.ds(s * RT, RT)], deg_hbm.at[c, pl.ds(s * RT, RT)])


CK = 64
NCH = EP // (NC * NS * CK)
SLOTS = 4
NGRP = NCH // SLOTS


@functools.partial(
    pl.kernel,
    out_type=jax.ShapeDtypeStruct((NC, NP, D), jnp.float32),
    mesh=_mesh,
    scratch_types=[
        pltpu.VMEM((CH, 128), jnp.int32),
        pltpu.VMEM((CK,), jnp.int32),
        pltpu.VMEM((CK,), jnp.int32),
        pltpu.VMEM((CK,), jnp.int32),
        pltpu.VMEM((CK,), jnp.int32),
        pltpu.VMEM((CK, D), jnp.float32),
        pltpu.VMEM((CK, D), jnp.float32),
        pltpu.VMEM((CK, D), jnp.float32),
        pltpu.VMEM((CK, D), jnp.float32),
        pltpu.VMEM_SHARED((NP, D), jnp.float32),
        pltpu.SemaphoreType.DMA,
        pltpu.SemaphoreType.DMA,
        pltpu.SemaphoreType.DMA,
        pltpu.SemaphoreType.DMA,
        pltpu.SemaphoreType.DMA,
        pltpu.SemaphoreType.DMA,
        pltpu.SemaphoreType.DMA,
        pltpu.SemaphoreType.DMA,
    ],
)
def _sc_agg(src_hbm, dst_hbm, y_hbm, zeros_hbm, agg_hbm,
            srcv, di0, di1, di2, di3, b0, b1, b2, b3, agg_sh,
            g0, g1, g2, g3, s0, s1, s2, s3):
    c = lax.axis_index("c")
    s = lax.axis_index("s")
    didx = (di0, di1, di2, di3)
    bufs = (b0, b1, b2, b3)
    gsem = (g0, g1, g2, g3)
    ssem = (s0, s1, s2, s3)
    pltpu.sync_copy(zeros_hbm.at[pl.ds(s * RT, RT)], agg_sh.at[pl.ds(s * RT, RT)])
    pltpu.sync_copy(src_hbm.at[c, s], srcv)
    plsc.subcore_barrier()

    def src_idx(t, k):
        return srcv.at[2 * t + k // 2, pl.ds((k % 2) * CK, CK)]

    def fire(t, k):
        pltpu.async_copy(dst_hbm.at[c, s, SLOTS * t + k], didx[k], gsem[k])
        pltpu.async_copy(y_hbm.at[src_idx(t, k)], bufs[k], gsem[k])

    def wait_gather(t, k):
        pltpu.make_async_copy(
            dst_hbm.at[c, s, SLOTS * t + k], didx[k], gsem[k]).wait()
        pltpu.make_async_copy(y_hbm.at[src_idx(t, k)], bufs[k], gsem[k]).wait()

    for k in range(SLOTS):
        fire(0, k)

    def body(t, carry):
        handles = []
        for k in range(SLOTS):
            wait_gather(t, k)
            handles.append(pltpu.async_copy(
                bufs[k], agg_sh.at[didx[k]], ssem[k], add=True))
        for k, h in enumerate(handles):
            h.wait()

            @pl.when(t < NGRP - 1)
            def _():
                fire(t + 1, k)

        return carry

    lax.fori_loop(0, NGRP, body, 0)
    plsc.subcore_barrier()
    pltpu.sync_copy(agg_sh.at[pl.ds(s * RT, RT)], agg_hbm.at[c, pl.ds(s * RT, RT)])


_BR = 1024


def _tc_y_body(x_ref, deg_ref, y_ref):
    dis = lax.rsqrt(deg_ref[0] + deg_ref[1] + 1.0)
    y_ref[...] = x_ref[...] * dis


def _tc_y(x_pad, deg2):
    return pl.pallas_call(
        _tc_y_body,
        grid=(NP // _BR,),
        in_specs=[
            pl.BlockSpec((_BR, D), lambda i: (i, 0)),
            pl.BlockSpec((NC, _BR, 1), lambda i: (0, i, 0)),
        ],
        out_specs=pl.BlockSpec((_BR, D), lambda i: (i, 0)),
        out_shape=jax.ShapeDtypeStruct((NP, D), jnp.float32),
    )(x_pad, deg2)


def _tc_out_body(agg_ref, x_ref, deg_ref, wt_ref, b_ref, o_ref):
    dis = lax.rsqrt(deg_ref[0] + deg_ref[1] + 1.0)
    t = (agg_ref[0] + agg_ref[1]) * dis + x_ref[...] * (dis * dis)
    o_ref[...] = (
        jnp.dot(t, wt_ref[...], preferred_element_type=jnp.float32) + b_ref[...]
    )


def _tc_out(agg2, x_pad, deg2, wt, b2):
    return pl.pallas_call(
        _tc_out_body,
        grid=(NP // _BR,),
        in_specs=[
            pl.BlockSpec((NC, _BR, D), lambda i: (0, i, 0)),
            pl.BlockSpec((_BR, D), lambda i: (i, 0)),
            pl.BlockSpec((NC, _BR, 1), lambda i: (0, i, 0)),
            pl.BlockSpec((D, D), lambda i: (0, 0)),
            pl.BlockSpec((1, D), lambda i: (0, 0)),
        ],
        out_specs=pl.BlockSpec((_BR, D), lambda i: (i, 0)),
        out_shape=jax.ShapeDtypeStruct((NP, D), jnp.float32),
    )(agg2, x_pad, deg2, wt, b2)


def kernel(x, edge_index, W, b):
    src = edge_index[0].astype(jnp.int32)
    dst = edge_index[1].astype(jnp.int32)
    pad = jnp.full((EP - E,), NP - 1, dtype=jnp.int32)
    src_p = jnp.concatenate([src, pad])
    dst_p = jnp.concatenate([dst, pad])
    src_r = src_p.reshape(NC, NS, CH, 128)
    dst_r = dst_p.reshape(NC, NS, NCH, CK)
    dst_deg = dst_p.reshape(NC, NS, CH, 128)
    x_pad = jnp.concatenate([x, jnp.zeros((NP - N, D), jnp.float32)])
    zeros_d = jnp.zeros((NP,), jnp.float32)
    ones_d = jnp.ones((128,), jnp.float32)
    zeros_f = jnp.zeros((NP, D), jnp.float32)

    deg2 = _sc_degree(dst_deg, ones_d, zeros_d).reshape(NC, NP, 1)
    y_pad = _tc_y(x_pad, deg2)
    agg2 = _sc_agg(src_r, dst_r, y_pad, zeros_f)
    out = _tc_out(agg2, x_pad, deg2, W.T, b.reshape(1, D))
    return out[:N]


# speedup vs baseline: 2.9983x; 2.9983x over previous
import functools

import jax
import jax.numpy as jnp
from jax import lax
from jax.experimental import pallas as pl
from jax.experimental.pallas import tpu as pltpu
from jax.experimental.pallas import tpu_sc as plsc

N = 10000
E = 320000
D = 128
NC = 2
NS = 16
NP = 10240
EP = 327680
CH = EP // (NC * NS * 128)
RT = NP // NS

_mesh = plsc.VectorSubcoreMesh(
    core_axis_name="c", subcore_axis_name="s", num_cores=NC, num_subcores=NS
)


@functools.partial(
    pl.kernel,
    out_type=jax.ShapeDtypeStruct((NC, NP), jnp.float32),
    mesh=_mesh,
    scratch_types=[
        pltpu.VMEM((CH, 128), jnp.int32),
        pltpu.VMEM((128,), jnp.float32),
        pltpu.VMEM_SHARED((NP,), jnp.float32),
        pltpu.SemaphoreType.DMA,
    ],
)
def _sc_degree(dst_hbm, ones_hbm, zeros_hbm, deg_hbm, dstv, ones_v, deg_sh, dsem):
    c = lax.axis_index("c")
    s = lax.axis_index("s")
    pltpu.sync_copy(zeros_hbm.at[pl.ds(s * RT, RT)], deg_sh.at[pl.ds(s * RT, RT)])
    pltpu.sync_copy(dst_hbm.at[c, s], dstv)
    pltpu.sync_copy(ones_hbm, ones_v)
    plsc.subcore_barrier()

    def fire(j, carry):
        pltpu.async_copy(ones_v, deg_sh.at[dstv.at[j]], dsem, add=True)
        return carry

    lax.fori_loop(0, CH, fire, 0)

    def drain(j, carry):
        pltpu.make_async_copy(ones_v, deg_sh.at[dstv.at[j]], dsem).wait()
        return carry

    lax.fori_loop(0, CH, drain, 0)
    plsc.subcore_barrier()
    pltpu.sync_copy(deg_sh.at[pl.ds(s * RT, RT)], deg_hbm.at[c, pl.ds(s * RT, RT)])


CK = 64
NCH = EP // (NC * NS * CK)
SLOTS = 4
NGRP = NCH // SLOTS


@functools.partial(
    pl.kernel,
    out_type=jax.ShapeDtypeStruct((NC, NP, D), jnp.float32),
    mesh=_mesh,
    scratch_types=[
        pltpu.VMEM((CH, 128), jnp.int32),
        pltpu.VMEM((CK,), jnp.int32),
        pltpu.VMEM((CK,), jnp.int32),
        pltpu.VMEM((CK,), jnp.int32),
        pltpu.VMEM((CK,), jnp.int32),
        pltpu.VMEM((CK, D), jnp.float32),
        pltpu.VMEM((CK, D), jnp.float32),
        pltpu.VMEM((CK, D), jnp.float32),
        pltpu.VMEM((CK, D), jnp.float32),
        pltpu.VMEM_SHARED((NP, D), jnp.float32),
        pltpu.SemaphoreType.DMA,
        pltpu.SemaphoreType.DMA,
        pltpu.SemaphoreType.DMA,
        pltpu.SemaphoreType.DMA,
        pltpu.SemaphoreType.DMA,
        pltpu.SemaphoreType.DMA,
        pltpu.SemaphoreType.DMA,
        pltpu.SemaphoreType.DMA,
    ],
)
def _sc_agg(src_hbm, dst_hbm, y_hbm, zeros_hbm, agg_hbm,
            srcv, di0, di1, di2, di3, b0, b1, b2, b3, agg_sh,
            g0, g1, g2, g3, s0, s1, s2, s3):
    c = lax.axis_index("c")
    s = lax.axis_index("s")
    didx = (di0, di1, di2, di3)
    bufs = (b0, b1, b2, b3)
    gsem = (g0, g1, g2, g3)
    ssem = (s0, s1, s2, s3)
    pltpu.sync_copy(zeros_hbm.at[pl.ds(s * RT, RT)], agg_sh.at[pl.ds(s * RT, RT)])
    pltpu.sync_copy(src_hbm.at[c, s], srcv)
    plsc.subcore_barrier()

    def src_idx(t, k):
        return srcv.at[2 * t + k // 2, pl.ds((k % 2) * CK, CK)]

    def fire(t, k):
        pltpu.async_copy(dst_hbm.at[c, s, SLOTS * t + k], didx[k], gsem[k])
        pltpu.async_copy(y_hbm.at[src_idx(t, k)], bufs[k], gsem[k])

    def wait_gather(t, k):
        pltpu.make_async_copy(
            dst_hbm.at[c, s, SLOTS * t + k], didx[k], gsem[k]).wait()
        pltpu.make_async_copy(y_hbm.at[src_idx(t, k)], bufs[k], gsem[k]).wait()

    for k in range(SLOTS):
        fire(0, k)

    def body(t, carry):
        handles = []
        for k in range(SLOTS):
            wait_gather(t, k)
            handles.append(pltpu.async_copy(
                bufs[k], agg_sh.at[didx[k]], ssem[k], add=True))
        for k, h in enumerate(handles):
            h.wait()

            @pl.when(t < NGRP - 1)
            def _():
                fire(t + 1, k)

        return carry

    lax.fori_loop(0, NGRP, body, 0)
    plsc.subcore_barrier()
    pltpu.sync_copy(agg_sh.at[pl.ds(s * RT, RT)], agg_hbm.at[c, pl.ds(s * RT, RT)])


_BR = 1024


def _tc_y_body(x_ref, deg_ref, y_ref):
    dis = lax.rsqrt(deg_ref[0] + deg_ref[1] + 1.0)
    y_ref[...] = x_ref[...] * dis


def _tc_y(x_pad, deg2):
    return pl.pallas_call(
        _tc_y_body,
        grid=(NP // _BR,),
        in_specs=[
            pl.BlockSpec((_BR, D), lambda i: (i, 0)),
            pl.BlockSpec((NC, _BR, 1), lambda i: (0, i, 0)),
        ],
        out_specs=pl.BlockSpec((_BR, D), lambda i: (i, 0)),
        out_shape=jax.ShapeDtypeStruct((NP, D), jnp.float32),
    )(x_pad, deg2)


def _tc_out_body(agg_ref, x_ref, deg_ref, wt_ref, b_ref, o_ref):
    dis = lax.rsqrt(deg_ref[0] + deg_ref[1] + 1.0)
    t = (agg_ref[0] + agg_ref[1]) * dis + x_ref[...] * (dis * dis)
    o_ref[...] = (
        jnp.dot(t, wt_ref[...], preferred_element_type=jnp.float32) + b_ref[...]
    )


def _tc_out(agg2, x_pad, deg2, wt, b2):
    return pl.pallas_call(
        _tc_out_body,
        grid=(NP // _BR,),
        in_specs=[
            pl.BlockSpec((NC, _BR, D), lambda i: (0, i, 0)),
            pl.BlockSpec((_BR, D), lambda i: (i, 0)),
            pl.BlockSpec((NC, _BR, 1), lambda i: (0, i, 0)),
            pl.BlockSpec((D, D), lambda i: (0, 0)),
            pl.BlockSpec((1, D), lambda i: (0, 0)),
        ],
        out_specs=pl.BlockSpec((_BR, D), lambda i: (i, 0)),
        out_shape=jax.ShapeDtypeStruct((NP, D), jnp.float32),
    )(agg2, x_pad, deg2, wt, b2)


def kernel(x, edge_index, W, b):
    src = edge_index[0].astype(jnp.int32)
    dst = edge_index[1].astype(jnp.int32)
    pad = N + (jnp.arange(EP - E, dtype=jnp.int32) % (NP - N))
    src_p = jnp.concatenate([src, pad])
    dst_p = jnp.concatenate([dst, pad])
    src_r = src_p.reshape(NC, NS, CH, 128)
    dst_r = dst_p.reshape(NC, NS, NCH, CK)
    dst_deg = dst_p.reshape(NC, NS, CH, 128)
    x_pad = jnp.concatenate([x, jnp.zeros((NP - N, D), jnp.float32)])
    zeros_d = jnp.zeros((NP,), jnp.float32)
    ones_d = jnp.ones((128,), jnp.float32)
    zeros_f = jnp.zeros((NP, D), jnp.float32)

    deg2 = _sc_degree(dst_deg, ones_d, zeros_d).reshape(NC, NP, 1)
    y_pad = _tc_y(x_pad, deg2)
    agg2 = _sc_agg(src_r, dst_r, y_pad, zeros_f)
    out = _tc_out(agg2, x_pad, deg2, W.T, b.reshape(1, D))
    return out[:N]
